# Initial kernel scaffold; baseline (speedup 1.0000x reference)
import jax
import jax.numpy as jnp
from jax.experimental import pallas as pl


def kernel(Is, Ispp, Il, line_thresh):
    raise NotImplementedError("write your pallas kernel here")


# speedup vs baseline: 36.7940x; 36.7940x over previous
import functools

import jax
import jax.numpy as jnp
from jax import lax
from jax.experimental import pallas as pl
from jax.experimental.pallas import tpu as pltpu
from jax.experimental.pallas import tpu_sc as plsc

B = 16
C = 4
HW = 512 * 512
NSEG = 1024
NC = 2
NS = 16
L = 16
HALF = HW // 2
CHUNK = 2048
NCHUNK = HALF // CHUNK
VPC = CHUNK // L
NF = 5
LANE_TAB = NSEG * NF
TAB = L * LANE_TAB

_mesh = plsc.VectorSubcoreMesh(
    core_axis_name="c", subcore_axis_name="s", num_cores=NC, num_subcores=NS
)
_params = pltpu.CompilerParams(needs_layout_passes=False)


def _iota16():
    return lax.iota(jnp.int32, L)


@functools.partial(
    pl.kernel,
    out_type=jax.ShapeDtypeStruct((B, C * NSEG), jnp.float32),
    mesh=_mesh,
    compiler_params=_params,
    scratch_types=[
        pltpu.VMEM((TAB,), jnp.float32),
        pltpu.VMEM((LANE_TAB,), jnp.float32),
        pltpu.VMEM((LANE_TAB,), jnp.float32),
        pltpu.VMEM((CHUNK,), jnp.int32),
        pltpu.VMEM((C * CHUNK,), jnp.float32),
        pltpu.VMEM((C * (NSEG // 2),), jnp.float32),
        pltpu.VMEM_SHARED((NS, LANE_TAB), jnp.float32),
    ],
)
def _seg_sums(is_hbm, lbl_hbm, avg_hbm, tab, comb, part, lblb, chb, avb, shr):
    s = lax.axis_index("s")
    c = lax.axis_index("c")
    b = c * (B // NC) + s // 2
    half = s % 2
    pix0 = half * HALF

    def _z(j, _):
        tab[pl.ds(j * L, L)] = jnp.zeros((L,), jnp.float32)
        return 0
    lax.fori_loop(0, TAB // L, _z, 0)

    lane_base = _iota16() * LANE_TAB
    ones = jnp.full((L,), 1.0, jnp.float32)

    def _chunk(g, _):
        base = pix0 + g * CHUNK
        pltpu.sync_copy(lbl_hbm.at[b, pl.ds(base, CHUNK)], lblb)
        for ch in range(C):
            pltpu.sync_copy(
                is_hbm.at[b, ch, pl.ds(base, CHUNK)],
                chb.at[pl.ds(ch * CHUNK, CHUNK)],
            )

        def _vreg(k, _):
            lbl = lblb[pl.ds(k * L, L)]
            idx0 = lane_base + lbl
            for ch in range(C):
                v = chb[pl.ds(ch * CHUNK + k * L, L)]
                plsc.addupdate_scatter(tab, [idx0 + ch * NSEG], v)
            plsc.addupdate_scatter(tab, [idx0 + C * NSEG], ones)
            return 0
        lax.fori_loop(0, VPC, _vreg, 0)
        return 0
    lax.fori_loop(0, NCHUNK, _chunk, 0)

    def _red(j, _):
        acc = tab[pl.ds(j * L, L)]
        for l in range(1, L):
            acc = acc + tab[pl.ds(l * LANE_TAB + j * L, L)]
        comb[pl.ds(j * L, L)] = acc
        return 0
    lax.fori_loop(0, LANE_TAB // L, _red, 0)

    pltpu.sync_copy(comb, shr.at[s])
    plsc.subcore_barrier()
    pltpu.sync_copy(shr.at[s ^ 1], part)

    def _add(j, _):
        comb[pl.ds(j * L, L)] = comb[pl.ds(j * L, L)] + part[pl.ds(j * L, L)]
        return 0
    lax.fori_loop(0, LANE_TAB // L, _add, 0)

    g0_half = half * (NSEG // 2)

    def _avg(v, _):
        g0 = g0_half + v * L
        n = comb[pl.ds(C * NSEG + g0, L)]
        nm = jnp.maximum(n, 1.0)
        glab = g0 + _iota16()
        for ch in range(C):
            a = comb[pl.ds(ch * NSEG + g0, L)] / nm
            a = jnp.where(glab == 0, 0.0, a)
            avb[pl.ds(ch * (NSEG // 2) + v * L, L)] = a
        return 0
    lax.fori_loop(0, (NSEG // 2) // L, _avg, 0)

    for ch in range(C):
        pltpu.sync_copy(
            avb.at[pl.ds(ch * (NSEG // 2), NSEG // 2)],
            avg_hbm.at[b, pl.ds(ch * NSEG + g0_half, NSEG // 2)],
        )


@functools.partial(
    pl.kernel,
    out_type=jax.ShapeDtypeStruct((NC * NS, L), jnp.float32),
    mesh=_mesh,
    compiler_params=_params,
    scratch_types=[
        pltpu.VMEM((C * NSEG,), jnp.float32),
        pltpu.VMEM((CHUNK,), jnp.int32),
        pltpu.VMEM((CHUNK,), jnp.int32),
        pltpu.VMEM((C * CHUNK,), jnp.float32),
        pltpu.VMEM((L,), jnp.float32),
        pltpu.VMEM((L,), jnp.float32),
    ],
)
def _loss(is_hbm, lbl_hbm, il_hbm, th_hbm, avg_hbm, out_hbm,
          avgv, lblb, ilb, chb, thb, accb):
    s = lax.axis_index("s")
    c = lax.axis_index("c")
    b = c * (B // NC) + s // 2
    half = s % 2
    pix0 = half * HALF
    row = c * NS + s

    pltpu.sync_copy(avg_hbm.at[b], avgv)
    pltpu.sync_copy(th_hbm, thb)
    tv = thb[...]

    def _chunk(g, acc):
        base = pix0 + g * CHUNK
        pltpu.sync_copy(lbl_hbm.at[b, pl.ds(base, CHUNK)], lblb)
        pltpu.sync_copy(il_hbm.at[b, pl.ds(base, CHUNK)], ilb)
        for ch in range(C):
            pltpu.sync_copy(
                is_hbm.at[b, ch, pl.ds(base, CHUNK)],
                chb.at[pl.ds(ch * CHUNK, CHUNK)],
            )

        def _vreg(k, a):
            lbl = lblb[pl.ds(k * L, L)]
            il = ilb[pl.ds(k * L, L)]
            nrm = jnp.zeros((L,), jnp.float32)
            for ch in range(C):
                v = chb[pl.ds(ch * CHUNK + k * L, L)]
                av = plsc.load_gather(avgv, [lbl + ch * NSEG])
                d = v - av
                nrm = nrm + d * d
            w = jnp.where(il.astype(jnp.float32) > tv, 1.0, 0.0)
            return a + w * nrm
        return lax.fori_loop(0, VPC, _vreg, acc)

    acc = lax.fori_loop(0, NCHUNK, _chunk, jnp.zeros((L,), jnp.float32))
    accb[...] = acc
    pltpu.sync_copy(accb, out_hbm.at[row])


def kernel(Is, Ispp, Il, line_thresh):
    is3 = Is.reshape(B, C, HW)
    lbl = Ispp.reshape(B, HW).astype(jnp.int32)
    il2 = Il.reshape(B, HW).astype(jnp.int32)
    th = jnp.full((L,), line_thresh, jnp.float32)
    avg = _seg_sums(is3, lbl)
    parts = _loss(is3, lbl, il2, th, avg)
    return jnp.sum(parts) / (B * HW)


# speedup vs baseline: 74.6939x; 2.0301x over previous
import functools

import jax
import jax.numpy as jnp
from jax import lax
from jax.experimental import pallas as pl
from jax.experimental.pallas import tpu as pltpu
from jax.experimental.pallas import tpu_sc as plsc

B = 16
C = 4
HW = 512 * 512
NSEG = 1024
NC = 2
NS = 16
L = 16
HALF = HW // 2

CHUNK1 = 2048
NCHUNK1 = HALF // CHUNK1
VPC1 = CHUNK1 // L
NF = 5
LANE_TAB = NSEG * NF
TAB = L * LANE_TAB

CHUNK2 = 8192
NCHUNK2 = HALF // CHUNK2
VPC2 = CHUNK2 // L

_mesh = plsc.VectorSubcoreMesh(
    core_axis_name="c", subcore_axis_name="s", num_cores=NC, num_subcores=NS
)
_params = pltpu.CompilerParams(needs_layout_passes=False)


def _iota16():
    return lax.iota(jnp.int32, L)


@functools.partial(
    pl.kernel,
    out_type=jax.ShapeDtypeStruct((B, C * NSEG), jnp.float32),
    mesh=_mesh,
    compiler_params=_params,
    scratch_types=[
        pltpu.VMEM((TAB,), jnp.float32),
        pltpu.VMEM((LANE_TAB,), jnp.float32),
        pltpu.VMEM((LANE_TAB,), jnp.float32),
        pltpu.VMEM((2 * CHUNK1,), jnp.int32),
        pltpu.VMEM((2 * C * CHUNK1,), jnp.float32),
        pltpu.VMEM((C * (NSEG // 2),), jnp.float32),
        pltpu.VMEM_SHARED((NS, LANE_TAB), jnp.float32),
        pltpu.SemaphoreType.DMA,
        pltpu.SemaphoreType.DMA,
    ],
)
def _seg_sums(is_hbm, lbl_hbm, avg_hbm, tab, comb, part, lblb, chb, avb, shr,
              sem0, sem1):
    s = lax.axis_index("s")
    c = lax.axis_index("c")
    b = c * (B // NC) + s // 2
    half = s % 2
    pix0 = half * HALF
    sems = (sem0, sem1)

    def _copies(g, p, sem):
        base = pix0 + g * CHUNK1
        cps = [pltpu.make_async_copy(
            lbl_hbm.at[b, pl.ds(base, CHUNK1)],
            lblb.at[pl.ds(p * CHUNK1, CHUNK1)], sem)]
        for ch in range(C):
            cps.append(pltpu.make_async_copy(
                is_hbm.at[b, ch, pl.ds(base, CHUNK1)],
                chb.at[pl.ds(p * C * CHUNK1 + ch * CHUNK1, CHUNK1)], sem))
        return cps

    def _z(j, _):
        tab[pl.ds(j * L, L)] = jnp.zeros((L,), jnp.float32)
        return 0
    lax.fori_loop(0, TAB // L, _z, 0)

    lane_base = _iota16() * LANE_TAB
    ones = jnp.full((L,), 1.0, jnp.float32)

    for p in range(2):
        for cp in _copies(p, p, sems[p]):
            cp.start()

    @pl.loop(0, NCHUNK1, step=2)
    def _pair(g):
        for p in range(2):
            gg = g + p
            for cp in _copies(gg, p, sems[p]):
                cp.wait()

            def _vreg(k, _):
                lbl = lblb[pl.ds(p * CHUNK1 + k * L, L)]
                idx0 = lane_base + lbl
                for ch in range(C):
                    v = chb[pl.ds(p * C * CHUNK1 + ch * CHUNK1 + k * L, L)]
                    plsc.addupdate_scatter(tab, [idx0 + ch * NSEG], v)
                plsc.addupdate_scatter(tab, [idx0 + C * NSEG], ones)
                return 0
            lax.fori_loop(0, VPC1, _vreg, 0)

            @pl.when(gg + 2 < NCHUNK1)
            def _():
                for cp in _copies(gg + 2, p, sems[p]):
                    cp.start()

    def _red(j, _):
        acc = tab[pl.ds(j * L, L)]
        for l in range(1, L):
            acc = acc + tab[pl.ds(l * LANE_TAB + j * L, L)]
        comb[pl.ds(j * L, L)] = acc
        return 0
    lax.fori_loop(0, LANE_TAB // L, _red, 0)

    pltpu.sync_copy(comb, shr.at[s])
    plsc.subcore_barrier()
    pltpu.sync_copy(shr.at[s ^ 1], part)

    def _add(j, _):
        comb[pl.ds(j * L, L)] = comb[pl.ds(j * L, L)] + part[pl.ds(j * L, L)]
        return 0
    lax.fori_loop(0, LANE_TAB // L, _add, 0)

    g0_half = half * (NSEG // 2)

    def _avg(v, _):
        g0 = g0_half + v * L
        n = comb[pl.ds(C * NSEG + g0, L)]
        nm = jnp.maximum(n, 1.0)
        glab = g0 + _iota16()
        for ch in range(C):
            a = comb[pl.ds(ch * NSEG + g0, L)] / nm
            a = jnp.where(glab == 0, 0.0, a)
            avb[pl.ds(ch * (NSEG // 2) + v * L, L)] = a
        return 0
    lax.fori_loop(0, (NSEG // 2) // L, _avg, 0)

    for ch in range(C):
        pltpu.sync_copy(
            avb.at[pl.ds(ch * (NSEG // 2), NSEG // 2)],
            avg_hbm.at[b, pl.ds(ch * NSEG + g0_half, NSEG // 2)],
        )


@functools.partial(
    pl.kernel,
    out_type=jax.ShapeDtypeStruct((NC * NS, L), jnp.float32),
    mesh=_mesh,
    compiler_params=_params,
    scratch_types=[
        pltpu.VMEM((C * NSEG,), jnp.float32),
        pltpu.VMEM((2 * CHUNK2,), jnp.int32),
        pltpu.VMEM((2 * CHUNK2,), jnp.int32),
        pltpu.VMEM((2 * C * CHUNK2,), jnp.float32),
        pltpu.VMEM((L,), jnp.float32),
        pltpu.VMEM((L,), jnp.float32),
        pltpu.SemaphoreType.DMA,
        pltpu.SemaphoreType.DMA,
    ],
)
def _loss(is_hbm, lbl_hbm, il_hbm, th_hbm, avg_hbm, out_hbm,
          avgv, lblb, ilb, chb, thb, accb, sem0, sem1):
    s = lax.axis_index("s")
    c = lax.axis_index("c")
    b = c * (B // NC) + s // 2
    half = s % 2
    pix0 = half * HALF
    row = c * NS + s
    sems = (sem0, sem1)

    def _copies(g, p, sem):
        base = pix0 + g * CHUNK2
        cps = [
            pltpu.make_async_copy(
                lbl_hbm.at[b, pl.ds(base, CHUNK2)],
                lblb.at[pl.ds(p * CHUNK2, CHUNK2)], sem),
            pltpu.make_async_copy(
                il_hbm.at[b, pl.ds(base, CHUNK2)],
                ilb.at[pl.ds(p * CHUNK2, CHUNK2)], sem),
        ]
        for ch in range(C):
            cps.append(pltpu.make_async_copy(
                is_hbm.at[b, ch, pl.ds(base, CHUNK2)],
                chb.at[pl.ds(p * C * CHUNK2 + ch * CHUNK2, CHUNK2)], sem))
        return cps

    pltpu.sync_copy(avg_hbm.at[b], avgv)
    pltpu.sync_copy(th_hbm, thb)
    tv = thb[...]

    for p in range(2):
        for cp in _copies(p, p, sems[p]):
            cp.start()

    def _pair(g, acc):
        for p in range(2):
            gg = g * 2 + p
            for cp in _copies(gg, p, sems[p]):
                cp.wait()

            def _vreg(k, a):
                lbl = lblb[pl.ds(p * CHUNK2 + k * L, L)]
                il = ilb[pl.ds(p * CHUNK2 + k * L, L)]
                nrm = jnp.zeros((L,), jnp.float32)
                for ch in range(C):
                    v = chb[pl.ds(p * C * CHUNK2 + ch * CHUNK2 + k * L, L)]
                    av = plsc.load_gather(avgv, [lbl + ch * NSEG])
                    d = v - av
                    nrm = nrm + d * d
                w = jnp.where(il.astype(jnp.float32) > tv, 1.0, 0.0)
                return a + w * nrm
            acc = lax.fori_loop(0, VPC2, _vreg, acc)

            @pl.when(gg + 2 < NCHUNK2)
            def _():
                for cp in _copies(gg + 2, p, sems[p]):
                    cp.start()
        return acc

    acc = lax.fori_loop(0, NCHUNK2 // 2, _pair,
                        jnp.zeros((L,), jnp.float32))
    accb[...] = acc
    pltpu.sync_copy(accb, out_hbm.at[row])


def kernel(Is, Ispp, Il, line_thresh):
    is3 = Is.reshape(B, C, HW)
    lbl = Ispp.reshape(B, HW).astype(jnp.int32)
    il2 = Il.reshape(B, HW).astype(jnp.int32)
    th = jnp.full((L,), line_thresh, jnp.float32)
    avg = _seg_sums(is3, lbl)
    parts = _loss(is3, lbl, il2, th, avg)
    return jnp.sum(parts) / (B * HW)


# speedup vs baseline: 80.9551x; 1.0838x over previous
import functools

import jax
import jax.numpy as jnp
from jax import lax
from jax.experimental import pallas as pl
from jax.experimental.pallas import tpu as pltpu
from jax.experimental.pallas import tpu_sc as plsc

B = 16
C = 4
HW = 512 * 512
NSEG = 1024
NC = 2
NS = 16
L = 16
HALF = HW // 2

CHUNK = 2048
NCHUNK = HALF // CHUNK
VPC = CHUNK // L
NF = 5
LANE_TAB = NSEG * NF
TAB = L * LANE_TAB
U = 4

_mesh = plsc.VectorSubcoreMesh(
    core_axis_name="c", subcore_axis_name="s", num_cores=NC, num_subcores=NS
)
_params = pltpu.CompilerParams(needs_layout_passes=False)


def _iota16():
    return lax.iota(jnp.int32, L)


@functools.partial(
    pl.kernel,
    out_type=jax.ShapeDtypeStruct((NC * NS, L), jnp.float32),
    mesh=_mesh,
    compiler_params=_params,
    scratch_types=[
        pltpu.VMEM((TAB,), jnp.float32),
        pltpu.VMEM((LANE_TAB,), jnp.float32),
        pltpu.VMEM((LANE_TAB,), jnp.float32),
        pltpu.VMEM((2 * CHUNK,), jnp.int32),
        pltpu.VMEM((2 * CHUNK,), jnp.int32),
        pltpu.VMEM((2 * C * CHUNK,), jnp.float32),
        pltpu.VMEM((L,), jnp.float32),
        pltpu.VMEM((L,), jnp.float32),
        pltpu.VMEM_SHARED((NS, LANE_TAB), jnp.float32),
        pltpu.SemaphoreType.DMA,
        pltpu.SemaphoreType.DMA,
    ],
)
def _superpixel(is_hbm, lbl_hbm, il_hbm, th_hbm, out_hbm,
                tab, comb, part, lblb, ilb, chb, thb, accb, shr, sem0, sem1):
    s = lax.axis_index("s")
    c = lax.axis_index("c")
    b = c * (B // NC) + s // 2
    half = s % 2
    pix0 = half * HALF
    row = c * NS + s
    sems = (sem0, sem1)

    def _copies1(g, p, sem):
        base = pix0 + g * CHUNK
        cps = [pltpu.make_async_copy(
            lbl_hbm.at[b, pl.ds(base, CHUNK)],
            lblb.at[pl.ds(p * CHUNK, CHUNK)], sem)]
        for ch in range(C):
            cps.append(pltpu.make_async_copy(
                is_hbm.at[b, ch, pl.ds(base, CHUNK)],
                chb.at[pl.ds((p * C + ch) * CHUNK, CHUNK)], sem))
        return cps

    def _copies2(g, p, sem):
        return _copies1(g, p, sem) + [pltpu.make_async_copy(
            il_hbm.at[b, pl.ds(pix0 + g * CHUNK, CHUNK)],
            ilb.at[pl.ds(p * CHUNK, CHUNK)], sem)]

    zero = jnp.zeros((L,), jnp.float32)

    def _z(j, _):
        for u in range(8):
            tab[pl.ds(j * 8 * L + u * L, L)] = zero
        return 0
    lax.fori_loop(0, TAB // (8 * L), _z, 0)

    lane_base = _iota16() * LANE_TAB
    ones = jnp.full((L,), 1.0, jnp.float32)

    for p in range(2):
        for cp in _copies1(p, p, sems[p]):
            cp.start()

    @pl.loop(0, NCHUNK, step=2)
    def _pair1(g):
        for p in range(2):
            gg = g + p
            for cp in _copies1(gg, p, sems[p]):
                cp.wait()

            def _vreg(kk, _):
                for u in range(U):
                    o = p * CHUNK + kk * U * L + u * L
                    lbl = lblb[pl.ds(o, L)]
                    idx0 = lane_base + lbl
                    for ch in range(C):
                        v = chb[pl.ds(p * C * CHUNK + ch * CHUNK
                                      + kk * U * L + u * L, L)]
                        plsc.addupdate_scatter(tab, [idx0 + ch * NSEG], v)
                    plsc.addupdate_scatter(tab, [idx0 + C * NSEG], ones)
                return 0
            lax.fori_loop(0, VPC // U, _vreg, 0)

            @pl.when(gg + 2 < NCHUNK)
            def _():
                for cp in _copies1(gg + 2, p, sems[p]):
                    cp.start()

    def _red(j, _):
        for u in range(2):
            o = (j * 2 + u) * L
            acc = tab[pl.ds(o, L)]
            for l in range(1, L):
                acc = acc + tab[pl.ds(l * LANE_TAB + o, L)]
            comb[pl.ds(o, L)] = acc
        return 0
    lax.fori_loop(0, LANE_TAB // (2 * L), _red, 0)

    pltpu.sync_copy(comb, shr.at[s])
    plsc.subcore_barrier()
    pltpu.sync_copy(shr.at[s ^ 1], part)

    def _add(j, _):
        for u in range(2):
            o = (j * 2 + u) * L
            comb[pl.ds(o, L)] = comb[pl.ds(o, L)] + part[pl.ds(o, L)]
        return 0
    lax.fori_loop(0, LANE_TAB // (2 * L), _add, 0)

    def _avg(v, _):
        g0 = v * L
        n = comb[pl.ds(C * NSEG + g0, L)]
        nm = jnp.maximum(n, 1.0)
        glab = g0 + _iota16()
        for ch in range(C):
            a = comb[pl.ds(ch * NSEG + g0, L)] / nm
            a = jnp.where(glab == 0, 0.0, a)
            comb[pl.ds(ch * NSEG + g0, L)] = a
        return 0
    lax.fori_loop(0, NSEG // L, _avg, 0)

    pltpu.sync_copy(th_hbm, thb)
    tv = thb[...]

    for p in range(2):
        for cp in _copies2(p, p, sems[p]):
            cp.start()

    def _pair2(g, acc):
        for p in range(2):
            gg = g * 2 + p
            for cp in _copies2(gg, p, sems[p]):
                cp.wait()

            def _vreg(kk, a):
                for u in range(U):
                    o = p * CHUNK + kk * U * L + u * L
                    lbl = lblb[pl.ds(o, L)]
                    il = ilb[pl.ds(o, L)]
                    nrm = zero
                    for ch in range(C):
                        v = chb[pl.ds(p * C * CHUNK + ch * CHUNK
                                      + kk * U * L + u * L, L)]
                        av = plsc.load_gather(comb, [lbl + ch * NSEG])
                        d = v - av
                        nrm = nrm + d * d
                    w = jnp.where(il.astype(jnp.float32) > tv, 1.0, 0.0)
                    a = a + w * nrm
                return a
            acc = lax.fori_loop(0, VPC // U, _vreg, acc)

            @pl.when(gg + 2 < NCHUNK)
            def _():
                for cp in _copies2(gg + 2, p, sems[p]):
                    cp.start()
        return acc

    acc = lax.fori_loop(0, NCHUNK // 2, _pair2, zero)
    accb[...] = acc
    pltpu.sync_copy(accb, out_hbm.at[row])


def kernel(Is, Ispp, Il, line_thresh):
    is3 = Is.reshape(B, C, HW)
    lbl = Ispp.reshape(B, HW).astype(jnp.int32)
    il2 = Il.reshape(B, HW).astype(jnp.int32)
    th = jnp.full((L,), line_thresh, jnp.float32)
    parts = _superpixel(is3, lbl, il2, th)
    return jnp.sum(parts) / (B * HW)


# speedup vs baseline: 105.4247x; 1.3023x over previous
import functools

import jax
import jax.numpy as jnp
from jax import lax
from jax.experimental import pallas as pl
from jax.experimental.pallas import tpu as pltpu
from jax.experimental.pallas import tpu_sc as plsc

B = 16
C = 4
HW = 512 * 512
NSEG = 1024
NC = 2
NS = 16
L = 16
HALF = HW // 2

CHUNK = 2048
NCHUNK = HALF // CHUNK
VPC = CHUNK // L
NF = 5
LANE_TAB = NSEG * NF
TAB = L * LANE_TAB
U = 4

_mesh = plsc.VectorSubcoreMesh(
    core_axis_name="c", subcore_axis_name="s", num_cores=NC, num_subcores=NS
)
_params = pltpu.CompilerParams(needs_layout_passes=False)


def _iota16():
    return lax.iota(jnp.int32, L)


@functools.partial(
    pl.kernel,
    out_type=jax.ShapeDtypeStruct((NC * NS, L), jnp.float32),
    mesh=_mesh,
    compiler_params=_params,
    scratch_types=[
        pltpu.VMEM((TAB,), jnp.float32),
        pltpu.VMEM((LANE_TAB,), jnp.float32),
        pltpu.VMEM((LANE_TAB,), jnp.float32),
        pltpu.VMEM((2 * CHUNK,), jnp.int32),
        pltpu.VMEM((2 * CHUNK,), jnp.int32),
        pltpu.VMEM((2 * C * CHUNK,), jnp.float32),
        pltpu.VMEM((L,), jnp.float32),
        pltpu.VMEM((L,), jnp.float32),
        pltpu.VMEM_SHARED((NS, LANE_TAB), jnp.float32),
        pltpu.SemaphoreType.DMA,
        pltpu.SemaphoreType.DMA,
    ],
)
def _superpixel(is_hbm, lbl_hbm, il_hbm, th_hbm, out_hbm,
                tab, comb, part, lblb, ilb, chb, thb, accb, shr, sem0, sem1):
    s = lax.axis_index("s")
    c = lax.axis_index("c")
    b = c * (B // NC) + s // 2
    half = s % 2
    pix0 = half * HALF
    row = c * NS + s
    sems = (sem0, sem1)

    def _copies1(g, p, sem):
        base = pix0 + g * CHUNK
        cps = [pltpu.make_async_copy(
            lbl_hbm.at[b, pl.ds(base, CHUNK)],
            lblb.at[pl.ds(p * CHUNK, CHUNK)], sem)]
        for ch in range(C):
            cps.append(pltpu.make_async_copy(
                is_hbm.at[b, ch, pl.ds(base, CHUNK)],
                chb.at[pl.ds((p * C + ch) * CHUNK, CHUNK)], sem))
        return cps

    def _copies2(g, p, sem):
        return _copies1(g, p, sem) + [pltpu.make_async_copy(
            il_hbm.at[b, pl.ds(pix0 + g * CHUNK, CHUNK)],
            ilb.at[pl.ds(p * CHUNK, CHUNK)], sem)]

    zero = jnp.zeros((L,), jnp.float32)

    def _z(j, _):
        for u in range(8):
            tab[pl.ds(j * 8 * L + u * L, L)] = zero
        return 0
    lax.fori_loop(0, TAB // (8 * L), _z, 0)

    lane_base = _iota16() * LANE_TAB
    ones = jnp.full((L,), 1.0, jnp.float32)

    for p in range(2):
        for cp in _copies1(p, p, sems[p]):
            cp.start()

    @pl.loop(0, NCHUNK, step=2)
    def _pair1(g):
        for p in range(2):
            gg = g + p
            for cp in _copies1(gg, p, sems[p]):
                cp.wait()

            @plsc.parallel_loop(0, VPC, unroll=U)
            def _vreg(k):
                o = p * CHUNK + k * L
                lbl = lblb[pl.ds(o, L)]
                idx0 = lane_base + lbl
                for ch in range(C):
                    v = chb[pl.ds(p * C * CHUNK + ch * CHUNK + k * L, L)]
                    plsc.addupdate_scatter(tab, [idx0 + ch * NSEG], v)
                plsc.addupdate_scatter(tab, [idx0 + C * NSEG], ones)

            @pl.when(gg + 2 < NCHUNK)
            def _():
                for cp in _copies1(gg + 2, p, sems[p]):
                    cp.start()

    def _red(j, _):
        for u in range(2):
            o = (j * 2 + u) * L
            acc = tab[pl.ds(o, L)]
            for l in range(1, L):
                acc = acc + tab[pl.ds(l * LANE_TAB + o, L)]
            comb[pl.ds(o, L)] = acc
        return 0
    lax.fori_loop(0, LANE_TAB // (2 * L), _red, 0)

    pltpu.sync_copy(comb, shr.at[s])
    plsc.subcore_barrier()
    pltpu.sync_copy(shr.at[s ^ 1], part)

    def _add(j, _):
        for u in range(2):
            o = (j * 2 + u) * L
            comb[pl.ds(o, L)] = comb[pl.ds(o, L)] + part[pl.ds(o, L)]
        return 0
    lax.fori_loop(0, LANE_TAB // (2 * L), _add, 0)

    def _avg(v, _):
        g0 = v * L
        n = comb[pl.ds(C * NSEG + g0, L)]
        nm = jnp.maximum(n, 1.0)
        glab = g0 + _iota16()
        for ch in range(C):
            a = comb[pl.ds(ch * NSEG + g0, L)] / nm
            a = jnp.where(glab == 0, 0.0, a)
            comb[pl.ds(ch * NSEG + g0, L)] = a
        return 0
    lax.fori_loop(0, NSEG // L, _avg, 0)

    pltpu.sync_copy(th_hbm, thb)
    tv = thb[...]

    for p in range(2):
        for cp in _copies2(p, p, sems[p]):
            cp.start()

    def _pair2(g, acc):
        for p in range(2):
            gg = g * 2 + p
            for cp in _copies2(gg, p, sems[p]):
                cp.wait()

            @plsc.parallel_loop(0, VPC, unroll=U, carry=acc)
            def _vreg(k, a):
                o = p * CHUNK + k * L
                lbl = lblb[pl.ds(o, L)]
                il = ilb[pl.ds(o, L)]
                nrm = zero
                for ch in range(C):
                    v = chb[pl.ds(p * C * CHUNK + ch * CHUNK + k * L, L)]
                    av = plsc.load_gather(comb, [lbl + ch * NSEG])
                    d = v - av
                    nrm = nrm + d * d
                w = jnp.where(il.astype(jnp.float32) > tv, 1.0, 0.0)
                return a + w * nrm
            acc = _vreg

            @pl.when(gg + 2 < NCHUNK)
            def _():
                for cp in _copies2(gg + 2, p, sems[p]):
                    cp.start()
        return acc

    acc = lax.fori_loop(0, NCHUNK // 2, _pair2, zero)
    accb[...] = acc
    pltpu.sync_copy(accb, out_hbm.at[row])


def kernel(Is, Ispp, Il, line_thresh):
    is3 = Is.reshape(B, C, HW)
    lbl = Ispp.reshape(B, HW)
    il2 = Il.reshape(B, HW)
    th = jnp.full((L,), line_thresh, jnp.float32)
    parts = _superpixel(is3, lbl, il2, th)
    return jnp.sum(parts) / (B * HW)


# speedup vs baseline: 164.5071x; 1.5604x over previous
import functools

import jax
import jax.numpy as jnp
from jax import lax
from jax.experimental import pallas as pl
from jax.experimental.pallas import tpu as pltpu
from jax.experimental.pallas import tpu_sc as plsc

B = 16
C = 4
H = 512
W = 512
HW = H * W
NSEG = 1024
NC = 2
NS = 16
L = 16
HROWS = H // 2

RPC = 4
CHUNK = RPC * W
NCHUNK = HROWS // RPC
VPR = W // L
NF = 5
LANE_TAB = NSEG * NF
TAB = L * LANE_TAB

_mesh = plsc.VectorSubcoreMesh(
    core_axis_name="c", subcore_axis_name="s", num_cores=NC, num_subcores=NS
)
_params = pltpu.CompilerParams(needs_layout_passes=False)


def _iota16():
    return lax.iota(jnp.int32, L)


@functools.partial(
    pl.kernel,
    out_type=jax.ShapeDtypeStruct((NC * NS, L), jnp.float32),
    mesh=_mesh,
    compiler_params=_params,
    scratch_types=[
        pltpu.VMEM((TAB,), jnp.float32),
        pltpu.VMEM((LANE_TAB,), jnp.float32),
        pltpu.VMEM((LANE_TAB,), jnp.float32),
        pltpu.VMEM((2 * RPC, W), jnp.int32),
        pltpu.VMEM((2 * RPC, W), jnp.int32),
        pltpu.VMEM((2 * C * RPC, W), jnp.float32),
        pltpu.VMEM((L,), jnp.float32),
        pltpu.VMEM((L,), jnp.float32),
        pltpu.VMEM_SHARED((NS, LANE_TAB), jnp.float32),
        pltpu.SemaphoreType.DMA,
        pltpu.SemaphoreType.DMA,
    ],
)
def _superpixel(is_hbm, lbl_hbm, il_hbm, th_hbm, out_hbm,
                tab, comb, part, lblb, ilb, chb, thb, accb, shr, sem0, sem1):
    s = lax.axis_index("s")
    c = lax.axis_index("c")
    b = c * (B // NC) + s // 2
    half = s % 2
    row0 = half * HROWS
    orow = c * NS + s
    sems = (sem0, sem1)

    def _copies1(g, p, sem):
        r0 = row0 + g * RPC
        cps = [pltpu.make_async_copy(
            lbl_hbm.at[b, 0, pl.ds(r0, RPC), :],
            lblb.at[pl.ds(p * RPC, RPC), :], sem)]
        for ch in range(C):
            cps.append(pltpu.make_async_copy(
                is_hbm.at[b, ch, pl.ds(r0, RPC), :],
                chb.at[pl.ds((p * C + ch) * RPC, RPC), :], sem))
        return cps

    def _copies2(g, p, sem):
        return _copies1(g, p, sem) + [pltpu.make_async_copy(
            il_hbm.at[b, pl.ds(row0 + g * RPC, RPC), :],
            ilb.at[pl.ds(p * RPC, RPC), :], sem)]

    zero = jnp.zeros((L,), jnp.float32)

    def _z(j, _):
        for u in range(8):
            tab[pl.ds(j * 8 * L + u * L, L)] = zero
        return 0
    lax.fori_loop(0, TAB // (8 * L), _z, 0)

    lane_base = _iota16() * LANE_TAB
    ones = jnp.full((L,), 1.0, jnp.float32)

    for p in range(2):
        for cp in _copies1(p, p, sems[p]):
            cp.start()

    @pl.loop(0, NCHUNK, step=2)
    def _pair1(g):
        for p in range(2):
            gg = g + p
            for cp in _copies1(gg, p, sems[p]):
                cp.wait()

            @plsc.parallel_loop(0, VPR, unroll=2)
            def _vreg(k):
                for r in range(RPC):
                    lbl = lblb[p * RPC + r, pl.ds(k * L, L)]
                    idx0 = lane_base + lbl
                    for ch in range(C):
                        v = chb[(p * C + ch) * RPC + r, pl.ds(k * L, L)]
                        plsc.addupdate_scatter(tab, [idx0 + ch * NSEG], v)
                    plsc.addupdate_scatter(tab, [idx0 + C * NSEG], ones)

            @pl.when(gg + 2 < NCHUNK)
            def _():
                for cp in _copies1(gg + 2, p, sems[p]):
                    cp.start()

    def _red(j, _):
        for u in range(2):
            o = (j * 2 + u) * L
            acc = tab[pl.ds(o, L)]
            for l in range(1, L):
                acc = acc + tab[pl.ds(l * LANE_TAB + o, L)]
            comb[pl.ds(o, L)] = acc
        return 0
    lax.fori_loop(0, LANE_TAB // (2 * L), _red, 0)

    pltpu.sync_copy(comb, shr.at[s])
    plsc.subcore_barrier()
    pltpu.sync_copy(shr.at[s ^ 1], part)

    def _add(j, _):
        for u in range(2):
            o = (j * 2 + u) * L
            comb[pl.ds(o, L)] = comb[pl.ds(o, L)] + part[pl.ds(o, L)]
        return 0
    lax.fori_loop(0, LANE_TAB // (2 * L), _add, 0)

    def _avg(v, _):
        g0 = v * L
        n = comb[pl.ds(C * NSEG + g0, L)]
        nm = jnp.maximum(n, 1.0)
        glab = g0 + _iota16()
        for ch in range(C):
            a = comb[pl.ds(ch * NSEG + g0, L)] / nm
            a = jnp.where(glab == 0, 0.0, a)
            comb[pl.ds(ch * NSEG + g0, L)] = a
        return 0
    lax.fori_loop(0, NSEG // L, _avg, 0)

    pltpu.sync_copy(th_hbm, thb)
    tv = thb[...]

    for p in range(2):
        for cp in _copies2(p, p, sems[p]):
            cp.start()

    def _pair2(g, acc):
        for p in range(2):
            gg = g * 2 + p
            for cp in _copies2(gg, p, sems[p]):
                cp.wait()

            @plsc.parallel_loop(0, VPR, unroll=2, carry=acc)
            def _vreg(k, a):
                for r in range(RPC):
                    lbl = lblb[p * RPC + r, pl.ds(k * L, L)]
                    il = ilb[p * RPC + r, pl.ds(k * L, L)]
                    nrm = zero
                    for ch in range(C):
                        v = chb[(p * C + ch) * RPC + r, pl.ds(k * L, L)]
                        av = plsc.load_gather(comb, [lbl + ch * NSEG])
                        d = v - av
                        nrm = nrm + d * d
                    w = jnp.where(il.astype(jnp.float32) > tv, 1.0, 0.0)
                    a = a + w * nrm
                return a
            acc = _vreg

            @pl.when(gg + 2 < NCHUNK)
            def _():
                for cp in _copies2(gg + 2, p, sems[p]):
                    cp.start()
        return acc

    acc = lax.fori_loop(0, NCHUNK // 2, _pair2, zero)
    accb[...] = acc
    pltpu.sync_copy(accb, out_hbm.at[orow])


def kernel(Is, Ispp, Il, line_thresh):
    th = jnp.full((L,), line_thresh, jnp.float32)
    parts = _superpixel(Is, Ispp, Il, th)
    return jnp.sum(parts) / (B * HW)


# speedup vs baseline: 175.5874x; 1.0674x over previous
import functools

import jax
import jax.numpy as jnp
from jax import lax
from jax.experimental import pallas as pl
from jax.experimental.pallas import tpu as pltpu
from jax.experimental.pallas import tpu_sc as plsc

B = 16
C = 4
H = 512
W = 512
HW = H * W
NSEG = 1024
NC = 2
NS = 16
L = 16
HROWS = H // 2

RPC = 4
CHUNK = RPC * W
NCHUNK = HROWS // RPC
VPR = W // L
NF = 5
LANE_TAB = NSEG * NF
TAB = L * LANE_TAB

_mesh = plsc.VectorSubcoreMesh(
    core_axis_name="c", subcore_axis_name="s", num_cores=NC, num_subcores=NS
)
_params = pltpu.CompilerParams(needs_layout_passes=False)


def _iota16():
    return lax.iota(jnp.int32, L)


@functools.partial(
    pl.kernel,
    out_type=jax.ShapeDtypeStruct((NC * NS, L), jnp.float32),
    mesh=_mesh,
    compiler_params=_params,
    scratch_types=[
        pltpu.VMEM((TAB,), jnp.float32),
        pltpu.VMEM((LANE_TAB,), jnp.float32),
        pltpu.VMEM((LANE_TAB,), jnp.float32),
        pltpu.VMEM((2 * RPC, W), jnp.int32),
        pltpu.VMEM((2 * RPC, W), jnp.int32),
        pltpu.VMEM((2 * C * RPC, W), jnp.float32),
        pltpu.VMEM((2 * NSEG,), jnp.int32),
        pltpu.VMEM((L,), jnp.float32),
        pltpu.VMEM((L,), jnp.float32),
        pltpu.VMEM_SHARED((NS, LANE_TAB), jnp.float32),
        pltpu.SemaphoreType.DMA,
        pltpu.SemaphoreType.DMA,
    ],
)
def _superpixel(is_hbm, lbl_hbm, il_hbm, th_hbm, out_hbm,
                tab, comb, part, lblb, ilb, chb, pckb, thb, accb, shr,
                sem0, sem1):
    s = lax.axis_index("s")
    c = lax.axis_index("c")
    b = c * (B // NC) + s // 2
    half = s % 2
    row0 = half * HROWS
    orow = c * NS + s
    sems = (sem0, sem1)

    def _copies1(g, p, sem):
        r0 = row0 + g * RPC
        cps = [pltpu.make_async_copy(
            lbl_hbm.at[b, 0, pl.ds(r0, RPC), :],
            lblb.at[pl.ds(p * RPC, RPC), :], sem)]
        for ch in range(C):
            cps.append(pltpu.make_async_copy(
                is_hbm.at[b, ch, pl.ds(r0, RPC), :],
                chb.at[pl.ds((p * C + ch) * RPC, RPC), :], sem))
        return cps

    def _copies2(g, p, sem):
        return _copies1(g, p, sem) + [pltpu.make_async_copy(
            il_hbm.at[b, pl.ds(row0 + g * RPC, RPC), :],
            ilb.at[pl.ds(p * RPC, RPC), :], sem)]

    zero = jnp.zeros((L,), jnp.float32)

    pltpu.sync_copy(th_hbm, thb)
    tv = thb[...]

    @plsc.parallel_loop(0, TAB // L, unroll=8)
    def _z(j):
        tab[pl.ds(j * L, L)] = zero

    lane_base = _iota16() * LANE_TAB
    ones = jnp.full((L,), 1.0, jnp.float32)

    for p in range(2):
        for cp in _copies1(p, p, sems[p]):
            cp.start()

    @pl.loop(0, NCHUNK, step=2)
    def _pair1(g):
        for p in range(2):
            gg = g + p
            for cp in _copies1(gg, p, sems[p]):
                cp.wait()

            @plsc.parallel_loop(0, VPR, unroll=2)
            def _vreg(k):
                for r in range(RPC):
                    lbl = lblb[p * RPC + r, pl.ds(k * L, L)]
                    idx0 = lane_base + lbl
                    for ch in range(C):
                        v = chb[(p * C + ch) * RPC + r, pl.ds(k * L, L)]
                        plsc.addupdate_scatter(tab, [idx0 + ch * NSEG], v)
                    plsc.addupdate_scatter(tab, [idx0 + C * NSEG], ones)

            @pl.when(gg + 2 < NCHUNK)
            def _():
                for cp in _copies1(gg + 2, p, sems[p]):
                    cp.start()

    @plsc.parallel_loop(0, LANE_TAB // L, unroll=2)
    def _red(j):
        o = j * L
        acc = tab[pl.ds(o, L)]
        for l in range(1, L):
            acc = acc + tab[pl.ds(l * LANE_TAB + o, L)]
        comb[pl.ds(o, L)] = acc

    pltpu.sync_copy(comb, shr.at[s])
    plsc.subcore_barrier()
    pltpu.sync_copy(shr.at[s ^ 1], part)

    @plsc.parallel_loop(0, LANE_TAB // L, unroll=2)
    def _add(j):
        o = j * L
        comb[pl.ds(o, L)] = comb[pl.ds(o, L)] + part[pl.ds(o, L)]

    def _bf16(a):
        u = plsc.bitcast(a, jnp.uint32)
        return (u + jnp.uint32(0x7FFF) + ((u >> 16) & jnp.uint32(1))) >> 16

    @plsc.parallel_loop(0, NSEG // L, unroll=2)
    def _avg(v):
        g0 = v * L
        n = comb[pl.ds(C * NSEG + g0, L)]
        inv = 1.0 / jnp.maximum(n, 1.0)
        keep = (g0 + _iota16()) != 0
        r = []
        for ch in range(C):
            a = comb[pl.ds(ch * NSEG + g0, L)] * inv
            a = jnp.where(keep, a, 0.0)
            r.append(_bf16(a))
        pckb[pl.ds(g0, L)] = plsc.bitcast((r[0] << 16) | r[1], jnp.int32)
        pckb[pl.ds(NSEG + g0, L)] = plsc.bitcast((r[2] << 16) | r[3],
                                                 jnp.int32)

    for p in range(2):
        for cp in _copies2(p, p, sems[p]):
            cp.start()

    def _pair2(g, acc):
        for p in range(2):
            gg = g * 2 + p
            for cp in _copies2(gg, p, sems[p]):
                cp.wait()

            @plsc.parallel_loop(0, VPR, unroll=2, carry=acc)
            def _vreg(k, a):
                hi = jnp.uint32(0xFFFF0000)
                for r in range(RPC):
                    lbl = lblb[p * RPC + r, pl.ds(k * L, L)]
                    il = ilb[p * RPC + r, pl.ds(k * L, L)]
                    q0 = plsc.bitcast(
                        plsc.load_gather(pckb, [lbl]), jnp.uint32)
                    q1 = plsc.bitcast(
                        plsc.load_gather(pckb, [lbl + NSEG]), jnp.uint32)
                    av = (plsc.bitcast(q0 & hi, jnp.float32),
                          plsc.bitcast(q0 << 16, jnp.float32),
                          plsc.bitcast(q1 & hi, jnp.float32),
                          plsc.bitcast(q1 << 16, jnp.float32))
                    nrm = zero
                    for ch in range(C):
                        v = chb[(p * C + ch) * RPC + r, pl.ds(k * L, L)]
                        d = v - av[ch]
                        nrm = nrm + d * d
                    w = jnp.where(il.astype(jnp.float32) > tv, 1.0, 0.0)
                    a = a + w * nrm
                return a
            acc = _vreg

            @pl.when(gg + 2 < NCHUNK)
            def _():
                for cp in _copies2(gg + 2, p, sems[p]):
                    cp.start()
        return acc

    acc = lax.fori_loop(0, NCHUNK // 2, _pair2, zero)
    accb[...] = acc
    pltpu.sync_copy(accb, out_hbm.at[orow])


def kernel(Is, Ispp, Il, line_thresh):
    th = jnp.full((L,), line_thresh, jnp.float32)
    parts = _superpixel(Is, Ispp, Il, th)
    return jnp.sum(parts) / (B * HW)


# speedup vs baseline: 198.1148x; 1.1283x over previous
import functools

import jax
import jax.numpy as jnp
from jax import lax
from jax.experimental import pallas as pl
from jax.experimental.pallas import tpu as pltpu
from jax.experimental.pallas import tpu_sc as plsc

B = 16
C = 4
H = 512
W = 512
HW = H * W
NSEG = 1024
NC = 2
NS = 16
L = 16
HROWS = H // 2

RPC = 4
CHUNK = RPC * W
NCHUNK = HROWS // RPC
VPR = W // L
NF = 5
LANE_TAB = NSEG * NF
TAB = L * LANE_TAB

_mesh = plsc.VectorSubcoreMesh(
    core_axis_name="c", subcore_axis_name="s", num_cores=NC, num_subcores=NS
)
_params = pltpu.CompilerParams(needs_layout_passes=False)


def _iota16():
    return lax.iota(jnp.int32, L)


@functools.partial(
    pl.kernel,
    out_type=jax.ShapeDtypeStruct((NC * NS, L), jnp.float32),
    mesh=_mesh,
    compiler_params=_params,
    scratch_types=[
        pltpu.VMEM((TAB,), jnp.float32),
        pltpu.VMEM((LANE_TAB,), jnp.float32),
        pltpu.VMEM((LANE_TAB,), jnp.float32),
        pltpu.VMEM((2 * RPC, W), jnp.int32),
        pltpu.VMEM((2 * RPC, W), jnp.int32),
        pltpu.VMEM((2 * C * RPC, W), jnp.float32),
        pltpu.VMEM((2 * NSEG,), jnp.int32),
        pltpu.VMEM((L,), jnp.float32),
        pltpu.VMEM((L,), jnp.float32),
        pltpu.VMEM_SHARED((NS, LANE_TAB), jnp.float32),
        pltpu.SemaphoreType.DMA,
        pltpu.SemaphoreType.DMA,
    ],
)
def _superpixel(is_hbm, lbl_hbm, il_hbm, th_hbm, out_hbm,
                tab, comb, part, lblb, ilb, chb, pckb, thb, accb, shr,
                sem0, sem1):
    s = lax.axis_index("s")
    c = lax.axis_index("c")
    b = c * (B // NC) + s // 2
    half = s % 2
    row0 = half * HROWS
    orow = c * NS + s
    sems = (sem0, sem1)

    def _copies1(g, p, sem):
        r0 = row0 + g * RPC
        cps = [pltpu.make_async_copy(
            lbl_hbm.at[b, 0, pl.ds(r0, RPC), :],
            lblb.at[pl.ds(p * RPC, RPC), :], sem)]
        for ch in range(C):
            cps.append(pltpu.make_async_copy(
                is_hbm.at[b, ch, pl.ds(r0, RPC), :],
                chb.at[pl.ds((p * C + ch) * RPC, RPC), :], sem))
        return cps

    def _copies2(g, p, sem):
        return _copies1(g, p, sem) + [pltpu.make_async_copy(
            il_hbm.at[b, pl.ds(row0 + g * RPC, RPC), :],
            ilb.at[pl.ds(p * RPC, RPC), :], sem)]

    zero = jnp.zeros((L,), jnp.float32)

    pltpu.sync_copy(th_hbm, thb)
    tv = thb[...]

    @plsc.parallel_loop(0, TAB // L, unroll=8)
    def _z(j):
        tab[pl.ds(j * L, L)] = zero

    iotav = _iota16()
    ones = jnp.full((L,), 1.0, jnp.float32)

    for p in range(2):
        for cp in _copies1(p, p, sems[p]):
            cp.start()

    @pl.loop(0, NCHUNK, step=2)
    def _pair1(g):
        for p in range(2):
            gg = g + p
            for cp in _copies1(gg, p, sems[p]):
                cp.wait()

            @plsc.parallel_loop(0, VPR, unroll=2)
            def _vreg(k):
                for r in range(RPC):
                    lbl = lblb[p * RPC + r, pl.ds(k * L, L)]
                    idx0 = (lbl << 4) + iotav
                    for ch in range(C):
                        v = chb[(p * C + ch) * RPC + r, pl.ds(k * L, L)]
                        plsc.addupdate_scatter(
                            tab, [idx0 + ch * (NSEG * L)], v)
                    plsc.addupdate_scatter(tab, [idx0 + C * (NSEG * L)],
                                           ones)

            @pl.when(gg + 2 < NCHUNK)
            def _():
                for cp in _copies1(gg + 2, p, sems[p]):
                    cp.start()

    last = iotav == (L - 1)

    @plsc.parallel_loop(0, LANE_TAB, unroll=4)
    def _red(e):
        cs = plsc.cumsum(tab[pl.ds(e * L, L)])
        plsc.store_scatter(comb, [iotav + (e - (L - 1))], cs, mask=last)

    pltpu.sync_copy(comb, shr.at[s])
    plsc.subcore_barrier()
    pltpu.sync_copy(shr.at[s ^ 1], part)

    @plsc.parallel_loop(0, LANE_TAB // L, unroll=2)
    def _add(j):
        o = j * L
        comb[pl.ds(o, L)] = comb[pl.ds(o, L)] + part[pl.ds(o, L)]

    def _bf16(a):
        u = plsc.bitcast(a, jnp.uint32)
        return (u + jnp.uint32(0x7FFF) + ((u >> 16) & jnp.uint32(1))) >> 16

    @plsc.parallel_loop(0, NSEG // L, unroll=2)
    def _avg(v):
        g0 = v * L
        n = comb[pl.ds(C * NSEG + g0, L)]
        inv = 1.0 / jnp.maximum(n, 1.0)
        keep = (g0 + _iota16()) != 0
        r = []
        for ch in range(C):
            a = comb[pl.ds(ch * NSEG + g0, L)] * inv
            a = jnp.where(keep, a, 0.0)
            r.append(_bf16(a))
        pckb[pl.ds(g0, L)] = plsc.bitcast((r[0] << 16) | r[1], jnp.int32)
        pckb[pl.ds(NSEG + g0, L)] = plsc.bitcast((r[2] << 16) | r[3],
                                                 jnp.int32)

    for p in range(2):
        for cp in _copies2(p, p, sems[p]):
            cp.start()

    def _pair2(g, acc):
        for p in range(2):
            gg = g * 2 + p
            for cp in _copies2(gg, p, sems[p]):
                cp.wait()

            @plsc.parallel_loop(0, VPR, unroll=2, carry=acc)
            def _vreg(k, a):
                hi = jnp.uint32(0xFFFF0000)
                for r in range(RPC):
                    lbl = lblb[p * RPC + r, pl.ds(k * L, L)]
                    il = ilb[p * RPC + r, pl.ds(k * L, L)]
                    q0 = plsc.bitcast(
                        plsc.load_gather(pckb, [lbl]), jnp.uint32)
                    q1 = plsc.bitcast(
                        plsc.load_gather(pckb, [lbl + NSEG]), jnp.uint32)
                    av = (plsc.bitcast(q0 & hi, jnp.float32),
                          plsc.bitcast(q0 << 16, jnp.float32),
                          plsc.bitcast(q1 & hi, jnp.float32),
                          plsc.bitcast(q1 << 16, jnp.float32))
                    nrm = zero
                    for ch in range(C):
                        v = chb[(p * C + ch) * RPC + r, pl.ds(k * L, L)]
                        d = v - av[ch]
                        nrm = nrm + d * d
                    w = jnp.where(il.astype(jnp.float32) > tv, 1.0, 0.0)
                    a = a + w * nrm
                return a
            acc = _vreg

            @pl.when(gg + 2 < NCHUNK)
            def _():
                for cp in _copies2(gg + 2, p, sems[p]):
                    cp.start()
        return acc

    acc = lax.fori_loop(0, NCHUNK // 2, _pair2, zero)
    accb[...] = acc
    pltpu.sync_copy(accb, out_hbm.at[orow])


def kernel(Is, Ispp, Il, line_thresh):
    th = jnp.full((L,), line_thresh, jnp.float32)
    parts = _superpixel(Is, Ispp, Il, th)
    return jnp.sum(parts) / (B * HW)


# speedup vs baseline: 199.5368x; 1.0072x over previous
import functools

import jax
import jax.numpy as jnp
from jax import lax
from jax.experimental import pallas as pl
from jax.experimental.pallas import tpu as pltpu
from jax.experimental.pallas import tpu_sc as plsc

B = 16
C = 4
H = 512
W = 512
HW = H * W
NSEG = 1024
NC = 2
NS = 16
L = 16
HROWS = H // 2

RPC = 4
CHUNK = RPC * W
NCHUNK = HROWS // RPC
VPR = W // L
NF = 5
LANE_TAB = NSEG * NF
TAB = L * LANE_TAB

_mesh = plsc.VectorSubcoreMesh(
    core_axis_name="c", subcore_axis_name="s", num_cores=NC, num_subcores=NS
)
_params = pltpu.CompilerParams(needs_layout_passes=False)


def _iota16():
    return lax.iota(jnp.int32, L)


@functools.partial(
    pl.kernel,
    out_type=jax.ShapeDtypeStruct((NC * NS, L), jnp.float32),
    mesh=_mesh,
    compiler_params=_params,
    scratch_types=[
        pltpu.VMEM((TAB,), jnp.float32),
        pltpu.VMEM((LANE_TAB,), jnp.float32),
        pltpu.VMEM((LANE_TAB,), jnp.float32),
        pltpu.VMEM((2 * RPC, W), jnp.int32),
        pltpu.VMEM((2 * RPC, W), jnp.int32),
        pltpu.VMEM((2 * C * RPC, W), jnp.float32),
        pltpu.VMEM((2 * NSEG,), jnp.int32),
        pltpu.VMEM((L,), jnp.float32),
        pltpu.VMEM((L,), jnp.float32),
        pltpu.VMEM_SHARED((NS, LANE_TAB), jnp.float32),
        pltpu.SemaphoreType.DMA,
        pltpu.SemaphoreType.DMA,
    ],
)
def _superpixel(is_hbm, lbl_hbm, il_hbm, th_hbm, out_hbm,
                tab, comb, part, lblb, ilb, chb, pckb, thb, accb, shr,
                sem0, sem1):
    s = lax.axis_index("s")
    c = lax.axis_index("c")
    b = c * (B // NC) + s // 2
    half = s % 2
    row0 = half * HROWS
    orow = c * NS + s
    sems = (sem0, sem1)

    def _copies1(g, p, sem):
        r0 = row0 + g * RPC
        cps = [pltpu.make_async_copy(
            lbl_hbm.at[b, 0, pl.ds(r0, RPC), :],
            lblb.at[pl.ds(p * RPC, RPC), :], sem)]
        for ch in range(C):
            cps.append(pltpu.make_async_copy(
                is_hbm.at[b, ch, pl.ds(r0, RPC), :],
                chb.at[pl.ds((p * C + ch) * RPC, RPC), :], sem))
        return cps

    def _copies2(g, p, sem):
        return _copies1(g, p, sem) + [pltpu.make_async_copy(
            il_hbm.at[b, pl.ds(row0 + g * RPC, RPC), :],
            ilb.at[pl.ds(p * RPC, RPC), :], sem)]

    zero = jnp.zeros((L,), jnp.float32)

    pltpu.sync_copy(th_hbm, thb)
    tv = thb[...]

    @plsc.parallel_loop(0, TAB // L, unroll=8)
    def _z(j):
        tab[pl.ds(j * L, L)] = zero

    iotav = _iota16()
    ones = jnp.full((L,), 1.0, jnp.float32)

    for p in range(2):
        for cp in _copies1(p, p, sems[p]):
            cp.start()

    @pl.loop(0, NCHUNK, step=2)
    def _pair1(g):
        for p in range(2):
            gg = g + p
            for cp in _copies1(gg, p, sems[p]):
                cp.wait()

            @plsc.parallel_loop(0, VPR, unroll=4)
            def _vreg(k):
                for r in range(RPC):
                    lbl = lblb[p * RPC + r, pl.ds(k * L, L)]
                    idx0 = (lbl << 4) + iotav
                    for ch in range(C):
                        v = chb[(p * C + ch) * RPC + r, pl.ds(k * L, L)]
                        plsc.addupdate_scatter(
                            tab, [idx0 + ch * (NSEG * L)], v)
                    plsc.addupdate_scatter(tab, [idx0 + C * (NSEG * L)],
                                           ones)

            @pl.when(gg + 2 < NCHUNK)
            def _():
                for cp in _copies1(gg + 2, p, sems[p]):
                    cp.start()

    last = iotav == (L - 1)

    @plsc.parallel_loop(0, LANE_TAB, unroll=4)
    def _red(e):
        cs = plsc.cumsum(tab[pl.ds(e * L, L)])
        plsc.store_scatter(comb, [iotav + (e - (L - 1))], cs, mask=last)

    pltpu.sync_copy(comb, shr.at[s])
    plsc.subcore_barrier()
    pltpu.sync_copy(shr.at[s ^ 1], part)

    @plsc.parallel_loop(0, LANE_TAB // L, unroll=2)
    def _add(j):
        o = j * L
        comb[pl.ds(o, L)] = comb[pl.ds(o, L)] + part[pl.ds(o, L)]

    def _bf16(a):
        u = plsc.bitcast(a, jnp.uint32)
        return (u + jnp.uint32(0x7FFF) + ((u >> 16) & jnp.uint32(1))) >> 16

    @plsc.parallel_loop(0, NSEG // L, unroll=2)
    def _avg(v):
        g0 = v * L
        n = comb[pl.ds(C * NSEG + g0, L)]
        inv = 1.0 / jnp.maximum(n, 1.0)
        keep = (g0 + _iota16()) != 0
        r = []
        for ch in range(C):
            a = comb[pl.ds(ch * NSEG + g0, L)] * inv
            a = jnp.where(keep, a, 0.0)
            r.append(_bf16(a))
        pckb[pl.ds(g0, L)] = plsc.bitcast((r[0] << 16) | r[1], jnp.int32)
        pckb[pl.ds(NSEG + g0, L)] = plsc.bitcast((r[2] << 16) | r[3],
                                                 jnp.int32)

    for p in range(2):
        for cp in _copies2(p, p, sems[p]):
            cp.start()

    def _pair2(g, acc):
        for p in range(2):
            gg = g * 2 + p
            for cp in _copies2(gg, p, sems[p]):
                cp.wait()

            @plsc.parallel_loop(0, VPR, unroll=2, carry=acc)
            def _vreg(k, a):
                hi = jnp.uint32(0xFFFF0000)
                for r in range(RPC):
                    lbl = lblb[p * RPC + r, pl.ds(k * L, L)]
                    il = ilb[p * RPC + r, pl.ds(k * L, L)]
                    q0 = plsc.bitcast(
                        plsc.load_gather(pckb, [lbl]), jnp.uint32)
                    q1 = plsc.bitcast(
                        plsc.load_gather(pckb, [lbl + NSEG]), jnp.uint32)
                    av = (plsc.bitcast(q0 & hi, jnp.float32),
                          plsc.bitcast(q0 << 16, jnp.float32),
                          plsc.bitcast(q1 & hi, jnp.float32),
                          plsc.bitcast(q1 << 16, jnp.float32))
                    nrm = zero
                    for ch in range(C):
                        v = chb[(p * C + ch) * RPC + r, pl.ds(k * L, L)]
                        d = v - av[ch]
                        nrm = nrm + d * d
                    w = jnp.where(il.astype(jnp.float32) > tv, 1.0, 0.0)
                    a = a + w * nrm
                return a
            acc = _vreg

            @pl.when(gg + 2 < NCHUNK)
            def _():
                for cp in _copies2(gg + 2, p, sems[p]):
                    cp.start()
        return acc

    acc = lax.fori_loop(0, NCHUNK // 2, _pair2, zero)
    accb[...] = acc
    pltpu.sync_copy(accb, out_hbm.at[orow])


def kernel(Is, Ispp, Il, line_thresh):
    th = jnp.full((L,), line_thresh, jnp.float32)
    parts = _superpixel(Is, Ispp, Il, th)
    return jnp.sum(parts) / (B * HW)
